# Initial kernel scaffold; baseline (speedup 1.0000x reference)
import jax
import jax.numpy as jnp
from jax.experimental import pallas as pl


def kernel(bond_embedding, node_coordinate, basic_attn, index_kj, index_ji, idx_i, idx_j, idx_k, angle_table, W_ain, b_ain, W_a1, b_a1, W_a2, b_a2, Wk, Wq, Wv, W_dis, W_ff1, W_ff2, g_ln, b_ln):
    raise NotImplementedError("write your pallas kernel here")


# speedup vs baseline: 5.5391x; 5.5391x over previous
import math
import functools
import jax
import jax.numpy as jnp
from jax.experimental import pallas as pl

_EDGE_DIM = 128
_NUM_HEAD = 8
_OUT_FEAT = 16
_CLASS_NUM = 6
_PI = 3.1415926


def _table_mlp_kernel(tbl_ref, wain_ref, bain_ref, wa2_ref, ba2_ref,
                      wa1_ref, ba1_ref, out_ref):
    ae = tbl_ref[...]
    ae = jnp.maximum(
        jnp.dot(ae, wain_ref[...].T, preferred_element_type=jnp.float32)
        + bain_ref[...], 0.0)
    ae = jnp.maximum(
        jnp.dot(ae, wa2_ref[...].T, preferred_element_type=jnp.float32)
        + ba2_ref[...], 0.0)
    ae = jnp.maximum(
        jnp.dot(ae, wa1_ref[...].T, preferred_element_type=jnp.float32)
        + ba1_ref[...], 0.0)
    out_ref[...] = ae


def _proj_kernel(bond_ref, ba_ref, wk_ref, wq_ref, wv_ref, wdis_ref,
                 k_ref, q_ref, v_ref, dd_ref):
    x = bond_ref[...]
    k_ref[...] = jnp.dot(x, wk_ref[...].T, preferred_element_type=jnp.float32)
    q_ref[...] = jnp.dot(x, wq_ref[...].T, preferred_element_type=jnp.float32)
    v_ref[...] = jnp.dot(x, wv_ref[...].T, preferred_element_type=jnp.float32)
    dd_ref[...] = ba_ref[...] * wdis_ref[...].reshape(1, _NUM_HEAD)


def _trip1_kernel(pi_ref, pj_ref, pk_ref, kg_ref, qg_ref, vg_ref, dd_ref,
                  tbl_ref, blk_ref, att_ref, vpre_ref):
    pji = pj_ref[...] - pi_ref[...]
    pki = pk_ref[...] - pi_ref[...]
    jx, jy, jz = pji[:, 0:1], pji[:, 1:2], pji[:, 2:3]
    kx, ky, kz = pki[:, 0:1], pki[:, 1:2], pki[:, 2:3]
    a = jx * kx + jy * ky + jz * kz
    cx = jy * kz - jz * ky
    cy = jz * kx - jx * kz
    cz = jx * ky - jy * kx
    b = jnp.sqrt(cx * cx + cy * cy + cz * cz)
    cls = jnp.zeros_like(a)
    for m in range(1, _CLASS_NUM):
        th = m * (_PI / _CLASS_NUM)
        cls = cls + jnp.where(b * math.cos(th) - a * math.sin(th) >= 0.0,
                              1.0, 0.0)
    onehot = jnp.where(
        cls.astype(jnp.int32)
        == jax.lax.broadcasted_iota(jnp.int32, (1, 8), 1), 1.0, 0.0)
    ae = jnp.dot(onehot, tbl_ref[...], preferred_element_type=jnp.float32)
    fk = kg_ref[...] + ae
    fq = qg_ref[...] + ae
    feat = fk * fq
    att = jnp.dot(feat, blk_ref[...],
                  preferred_element_type=jnp.float32) * 0.25 + dd_ref[...]
    att_ref[...] = jnp.exp(att)
    vpre_ref[...] = vg_ref[...] + ae


def _trip2_kernel(att_ref, rec_ref, vpre_ref, blk_ref, out_ref):
    w = att_ref[...] / rec_ref[...]
    w128 = jnp.dot(w, blk_ref[...].T, preferred_element_type=jnp.float32)
    out_ref[...] = vpre_ref[...] * w128


def _ffn_kernel(he_ref, g_ref, b_ref, w1_ref, w2_ref, out_ref):
    he = he_ref[...]
    mu = jnp.mean(he, axis=-1, keepdims=True)
    var = jnp.mean((he - mu) ** 2, axis=-1, keepdims=True)
    h = (he - mu) / jnp.sqrt(var + 1e-5) * g_ref[...] + b_ref[...]
    h = jnp.maximum(
        jnp.dot(h, w1_ref[...].T, preferred_element_type=jnp.float32), 0.0)
    h = jnp.dot(h, w2_ref[...].T, preferred_element_type=jnp.float32)
    h = h + he
    mu = jnp.mean(h, axis=-1, keepdims=True)
    var = jnp.mean((h - mu) ** 2, axis=-1, keepdims=True)
    out_ref[...] = (h - mu) / jnp.sqrt(var + 1e-5)


def kernel(bond_embedding, node_coordinate, basic_attn, index_kj, index_ji,
           idx_i, idx_j, idx_k, angle_table, W_ain, b_ain, W_a1, b_a1,
           W_a2, b_a2, Wk, Wq, Wv, W_dis, W_ff1, W_ff2, g_ln, b_ln):
    E, D = bond_embedding.shape
    T = index_kj.shape[0]
    H = _NUM_HEAD

    tbl8 = jnp.zeros((8, D), jnp.float32).at[:_CLASS_NUM].set(angle_table)
    table_mlp = pl.pallas_call(
        _table_mlp_kernel,
        out_shape=jax.ShapeDtypeStruct((8, D), jnp.float32),
    )(tbl8, W_ain, b_ain.reshape(1, D), W_a2, b_a2.reshape(1, D),
      W_a1, b_a1.reshape(1, D))

    RB = 1600
    ge = E // RB
    K, Q, V, dval = pl.pallas_call(
        _proj_kernel,
        grid=(ge,),
        in_specs=[
            pl.BlockSpec((RB, D), lambda i: (i, 0)),
            pl.BlockSpec((RB, 1), lambda i: (i, 0)),
            pl.BlockSpec((D, D), lambda i: (0, 0)),
            pl.BlockSpec((D, D), lambda i: (0, 0)),
            pl.BlockSpec((D, D), lambda i: (0, 0)),
            pl.BlockSpec((H, 1), lambda i: (0, 0)),
        ],
        out_specs=[
            pl.BlockSpec((RB, D), lambda i: (i, 0)),
            pl.BlockSpec((RB, D), lambda i: (i, 0)),
            pl.BlockSpec((RB, D), lambda i: (i, 0)),
            pl.BlockSpec((RB, H), lambda i: (i, 0)),
        ],
        out_shape=[
            jax.ShapeDtypeStruct((E, D), jnp.float32),
            jax.ShapeDtypeStruct((E, D), jnp.float32),
            jax.ShapeDtypeStruct((E, D), jnp.float32),
            jax.ShapeDtypeStruct((E, H), jnp.float32),
        ],
    )(bond_embedding, basic_attn, Wk, Wq, Wv, W_dis)

    coords8 = jnp.zeros((node_coordinate.shape[0], 8), jnp.float32)
    coords8 = coords8.at[:, :3].set(node_coordinate)
    pos_i = coords8[idx_i]
    pos_j = coords8[idx_j]
    pos_k = coords8[idx_k]
    Kg = K[index_kj]
    Qg = Q[index_ji]
    Vg = V[index_kj]
    ddg = dval[index_kj]

    blkdiag = jnp.where(
        (jax.lax.broadcasted_iota(jnp.int32, (D, H), 0) // _OUT_FEAT)
        == jax.lax.broadcasted_iota(jnp.int32, (D, H), 1),
        1.0, 0.0).astype(jnp.float32)

    SB = 1600
    gt = T // SB
    att_decay, v_pre = pl.pallas_call(
        _trip1_kernel,
        grid=(gt,),
        in_specs=[
            pl.BlockSpec((SB, 8), lambda i: (i, 0)),
            pl.BlockSpec((SB, 8), lambda i: (i, 0)),
            pl.BlockSpec((SB, 8), lambda i: (i, 0)),
            pl.BlockSpec((SB, D), lambda i: (i, 0)),
            pl.BlockSpec((SB, D), lambda i: (i, 0)),
            pl.BlockSpec((SB, D), lambda i: (i, 0)),
            pl.BlockSpec((SB, H), lambda i: (i, 0)),
            pl.BlockSpec((8, D), lambda i: (0, 0)),
            pl.BlockSpec((D, H), lambda i: (0, 0)),
        ],
        out_specs=[
            pl.BlockSpec((SB, H), lambda i: (i, 0)),
            pl.BlockSpec((SB, D), lambda i: (i, 0)),
        ],
        out_shape=[
            jax.ShapeDtypeStruct((T, H), jnp.float32),
            jax.ShapeDtypeStruct((T, D), jnp.float32),
        ],
    )(pos_i, pos_j, pos_k, Kg, Qg, Vg, ddg, table_mlp, blkdiag)

    att_all = jnp.zeros((E, H), jnp.float32).at[index_ji].add(att_decay)
    att_all_g = att_all[index_ji]

    v_att = pl.pallas_call(
        _trip2_kernel,
        grid=(gt,),
        in_specs=[
            pl.BlockSpec((SB, H), lambda i: (i, 0)),
            pl.BlockSpec((SB, H), lambda i: (i, 0)),
            pl.BlockSpec((SB, D), lambda i: (i, 0)),
            pl.BlockSpec((D, H), lambda i: (0, 0)),
        ],
        out_specs=pl.BlockSpec((SB, D), lambda i: (i, 0)),
        out_shape=jax.ShapeDtypeStruct((T, D), jnp.float32),
    )(att_decay, att_all_g, v_pre, blkdiag)

    he = bond_embedding.at[index_ji].add(v_att)

    out = pl.pallas_call(
        _ffn_kernel,
        grid=(ge,),
        in_specs=[
            pl.BlockSpec((RB, D), lambda i: (i, 0)),
            pl.BlockSpec((1, D), lambda i: (0, 0)),
            pl.BlockSpec((1, D), lambda i: (0, 0)),
            pl.BlockSpec((2 * D, D), lambda i: (0, 0)),
            pl.BlockSpec((D, 2 * D), lambda i: (0, 0)),
        ],
        out_specs=pl.BlockSpec((RB, D), lambda i: (i, 0)),
        out_shape=jax.ShapeDtypeStruct((E, D), jnp.float32),
    )(he, g_ln.reshape(1, D), b_ln.reshape(1, D), W_ff1, W_ff2)
    return out
